# baseline (speedup 1.0000x reference)
import functools

import jax
import jax.numpy as jnp
from jax import lax
from jax.experimental import pallas as pl
from jax.experimental.pallas import tpu as pltpu
from jax.experimental.pallas import tpu_sc as plsc

S = 2048
H = 1024
E = 8
K = 2
CAP = 768
THRESH = 0.5
SB = 256
NS = 16
LANES = 16
TPB = S // NS
BIG = 1 << 20


KC = 256


def _cdot(a, b):
    nk = a.shape[1] // KC
    acc = jnp.dot(a[:, :KC], b[:KC, :], preferred_element_type=jnp.float32)
    for k in range(1, nk):
        acc = acc + jnp.dot(a[:, k * KC:(k + 1) * KC],
                            b[k * KC:(k + 1) * KC, :],
                            preferred_element_type=jnp.float32)
    return acc


def _tc1_body(h_ref, rw1_ref, rb1_ref, rw2_ref, rb2_ref,
              iw1_ref, ib1_ref, iw2_ref, ib2_ref,
              probs_ref, imp_ref, psum_ref, acc_ref):
    i = pl.program_id(0)
    nsteps = pl.num_programs(0)

    @pl.when(i == 0)
    def _init():
        acc_ref[...] = jnp.zeros_like(acc_ref)

    h = h_ref[...]
    rh = jnp.maximum(_cdot(h, rw1_ref[...]) + rb1_ref[...], 0.0)
    logits = _cdot(rh, rw2_ref[...]) + rb2_ref[...]
    m = jnp.max(logits, axis=1, keepdims=True)
    ex = jnp.exp(logits - m)
    probs = ex / jnp.sum(ex, axis=1, keepdims=True)
    probs_ref[...] = probs

    ih = jnp.maximum(_cdot(h, iw1_ref[...]) + ib1_ref[...], 0.0)
    il = _cdot(ih, iw2_ref[...]) + ib2_ref[...]
    imp_ref[...] = jax.nn.sigmoid(il)

    acc_ref[...] = acc_ref[...] + jnp.sum(probs, axis=0, keepdims=True)

    @pl.when(i == nsteps - 1)
    def _fin():
        psum_ref[...] = acc_ref[...]


def _run_tc1(h2, r_w1, r_b1, r_w2, r_b2, imp_w1, imp_b1, imp_w2, imp_b2):
    n = h2.shape[0]
    grid = n // SB
    full = lambda *shape: pl.BlockSpec(shape, lambda i: (0,) * len(shape))
    return pl.pallas_call(
        _tc1_body,
        grid=(grid,),
        in_specs=[
            pl.BlockSpec((SB, H), lambda i: (i, 0)),
            full(H, H), full(1, H), full(H, E), full(1, E),
            full(H, H // 2), full(1, H // 2), full(H // 2, 1), full(1, 1),
        ],
        out_specs=[
            pl.BlockSpec((SB, E), lambda i: (i, 0)),
            pl.BlockSpec((SB, 1), lambda i: (i, 0)),
            pl.BlockSpec((1, E), lambda i: (0, 0)),
        ],
        out_shape=(
            jax.ShapeDtypeStruct((n, E), jnp.float32),
            jax.ShapeDtypeStruct((n, 1), jnp.float32),
            jax.ShapeDtypeStruct((1, E), jnp.float32),
        ),
        scratch_shapes=[pltpu.VMEM((1, E), jnp.float32)],
    )(h2, r_w1, r_b1.reshape(1, H), r_w2, r_b2.reshape(1, E),
      imp_w1, imp_b1.reshape(1, H // 2), imp_w2, imp_b2.reshape(1, 1))


def _sc_route_body(probs_hbm, posk_hbm, pval_hbm, cnt_hbm,
                   probs_v, posk_v, pval_v, cnt_v):
    sid = lax.axis_index("s")
    base = sid * (TPB * E)
    pltpu.sync_copy(probs_hbm.at[pl.ds(base, TPB * E)], probs_v)
    lane = lax.iota(jnp.int32, LANES)

    cnt = [jnp.int32(0)] * E
    for g in range(TPB // LANES):
        idx0 = lane * E + g * LANES * E
        vs = [plsc.load_gather(probs_v, [idx0 + e]) for e in range(E)]
        m1 = vs[0]
        i1 = jnp.zeros((LANES,), jnp.int32)
        for e in range(1, E):
            b = vs[e] > m1
            i1 = jnp.where(b, jnp.int32(e), i1)
            m1 = jnp.where(b, vs[e], m1)
        m2 = jnp.full((LANES,), -1.0, jnp.float32)
        i2 = jnp.zeros((LANES,), jnp.int32)
        for e in range(E):
            valid = (i1 != e) & (vs[e] > m2)
            i2 = jnp.where(valid, jnp.int32(e), i2)
            m2 = jnp.where(valid, vs[e], m2)
        denom = m1 + m2 + 1e-8
        p1 = m1 / denom
        p2 = m2 / denom
        for e in range(E):
            s1 = i1 == e
            s2 = i2 == e
            sel = s1 | s2
            sel_i = sel.astype(jnp.int32)
            inc = plsc.cumsum(sel_i)
            pos = inc - sel_i + cnt[e]
            cnt[e] = cnt[e] + jnp.sum(sel_i)
            val = jnp.where(s1, p1, 0.0) + jnp.where(s2, p2, 0.0)
            poss = jnp.where(sel, pos, jnp.int32(BIG))
            plsc.store_scatter(posk_v, [idx0 + e], poss)
            plsc.store_scatter(pval_v, [idx0 + e], val)

    cv = jnp.zeros((LANES,), jnp.int32)
    for e in range(E):
        cv = jnp.where(lane == e, cnt[e], cv)
    cnt_v[...] = cv
    pltpu.sync_copy(cnt_v, cnt_hbm.at[sid])
    pltpu.sync_copy(posk_v, posk_hbm.at[pl.ds(base, TPB * E)])
    pltpu.sync_copy(pval_v, pval_hbm.at[pl.ds(base, TPB * E)])


def _run_sc_route(probs_flat):
    mesh = plsc.VectorSubcoreMesh(core_axis_name="c", subcore_axis_name="s",
                                  num_cores=1)
    k = pl.kernel(
        _sc_route_body,
        out_type=(
            jax.ShapeDtypeStruct((S * E,), jnp.int32),
            jax.ShapeDtypeStruct((S * E,), jnp.float32),
            jax.ShapeDtypeStruct((NS, LANES), jnp.int32),
        ),
        mesh=mesh,
        scratch_types=[
            pltpu.VMEM((TPB * E,), jnp.float32),
            pltpu.VMEM((TPB * E,), jnp.int32),
            pltpu.VMEM((TPB * E,), jnp.float32),
            pltpu.VMEM((LANES,), jnp.int32),
        ],
        compiler_params=pltpu.CompilerParams(needs_layout_passes=False),
    )
    return k(probs_flat)


def _tc2_body(posk_ref, pval_ref, imp_ref, psum_ref, cnt_ref,
              disp_ref, comb_ref, aux_ref):
    i = pl.program_id(0)
    nsteps = pl.num_programs(0)

    cnt_f = cnt_ref[...].astype(jnp.float32)
    off0 = jnp.zeros((1, LANES), jnp.float32)
    off1 = jnp.zeros((1, LANES), jnp.float32)
    for w in range(NS):
        rw = cnt_f[w:w + 1, :]
        off0 = off0 + rw * (w < 2 * i).astype(jnp.float32)
        off1 = off1 + rw * (w < 2 * i + 1).astype(jnp.float32)
    row = jax.lax.broadcasted_iota(jnp.int32, (SB, 1), 0)
    off_be = jnp.where(row < TPB, off0[:, 0:E], off1[:, 0:E])

    p = posk_ref[...] + off_be.astype(jnp.int32)
    posk = jnp.where(p < CAP, p, -1)
    factor = 1.0 + (imp_ref[...] > THRESH).astype(jnp.float32)
    cap_iota = jax.lax.broadcasted_iota(jnp.int32, (SB, E, CAP), 2)
    hit = (cap_iota == posk[:, :, None]).astype(jnp.float32)
    disp_ref[...] = hit
    comb_ref[...] = hit * (pval_ref[...] * factor)[:, :, None]

    @pl.when(i == nsteps - 1)
    def _fin():
        prob_mean = psum_ref[...] / S
        usage = jnp.sum(cnt_f, axis=0, keepdims=True)[0:1, 0:E] / (S * K)
        aux_ref[...] = jnp.sum(prob_mean * usage,
                               keepdims=True).reshape(1, 1) * E


def _run_tc2(posk, pval, imp, psum, cnt):
    n = posk.shape[0]
    grid = n // SB
    return pl.pallas_call(
        _tc2_body,
        grid=(grid,),
        in_specs=[
            pl.BlockSpec((SB, E), lambda i: (i, 0)),
            pl.BlockSpec((SB, E), lambda i: (i, 0)),
            pl.BlockSpec((SB, 1), lambda i: (i, 0)),
            pl.BlockSpec((1, E), lambda i: (0, 0)),
            pl.BlockSpec((NS, LANES), lambda i: (0, 0)),
        ],
        out_specs=[
            pl.BlockSpec((SB, E, CAP), lambda i: (i, 0, 0)),
            pl.BlockSpec((SB, E, CAP), lambda i: (i, 0, 0)),
            pl.BlockSpec((1, 1), lambda i: (0, 0)),
        ],
        out_shape=(
            jax.ShapeDtypeStruct((n, E, CAP), jnp.float32),
            jax.ShapeDtypeStruct((n, E, CAP), jnp.float32),
            jax.ShapeDtypeStruct((1, 1), jnp.float32),
        ),
    )(posk, pval, imp, psum, cnt)


def kernel(hidden_states, r_w1, r_b1, r_w2, r_b2,
           imp_w1, imp_b1, imp_w2, imp_b2):
    B = hidden_states.shape[0]
    h2 = hidden_states.reshape(B * S, H)

    probs, imp, psum = _run_tc1(h2, r_w1, r_b1, r_w2, r_b2,
                                imp_w1, imp_b1, imp_w2, imp_b2)
    posk_flat, pval_flat, cnt = _run_sc_route(probs.reshape(B * S * E))
    posk = posk_flat.reshape(B * S, E)
    pval = pval_flat.reshape(B * S, E)
    disp, comb, aux = _run_tc2(posk, pval, imp, psum, cnt)

    dispatch = disp.reshape(B, S, E, CAP)
    combine = comb.reshape(B, S, E, CAP)
    router_probs = probs.reshape(B, S, E)
    importance = imp.reshape(B, S)
    aux_loss = aux.reshape(())
    return (dispatch, combine, router_probs, aux_loss, importance)


# speedup vs baseline: 1.6066x; 1.6066x over previous
import jax
import jax.numpy as jnp
from jax.experimental import pallas as pl
from jax.experimental.pallas import tpu as pltpu

S = 2048
H = 1024
E = 8
K = 2
CAP = 768
THRESH = 0.5
SB = 256
KC = 256


def _cdot(a, b):
    nk = a.shape[1] // KC
    acc = jnp.dot(a[:, :KC], b[:KC, :], preferred_element_type=jnp.float32)
    for k in range(1, nk):
        acc = acc + jnp.dot(a[:, k * KC:(k + 1) * KC],
                            b[k * KC:(k + 1) * KC, :],
                            preferred_element_type=jnp.float32)
    return acc


def _router_body(h_ref, rw1_ref, rb1_ref, rw2_ref, rb2_ref,
                 iw1_ref, ib1_ref, iw2_ref, ib2_ref,
                 disp_ref, comb_ref, probs_ref, imp_ref, aux_ref,
                 cnt_ref, psum_ref, usum_ref):
    i = pl.program_id(0)
    nsteps = pl.num_programs(0)

    @pl.when(i == 0)
    def _init():
        cnt_ref[...] = jnp.zeros_like(cnt_ref)
        psum_ref[...] = jnp.zeros_like(psum_ref)
        usum_ref[...] = jnp.zeros_like(usum_ref)

    h = h_ref[...]

    rh = jnp.maximum(_cdot(h, rw1_ref[...]) + rb1_ref[...], 0.0)
    logits = _cdot(rh, rw2_ref[...]) + rb2_ref[...]

    m = jnp.max(logits, axis=1, keepdims=True)
    ex = jnp.exp(logits - m)
    probs = ex / jnp.sum(ex, axis=1, keepdims=True)
    probs_ref[...] = probs

    eidx = jax.lax.broadcasted_iota(jnp.int32, (SB, E), 1)
    m1 = jnp.max(probs, axis=1, keepdims=True)
    i1 = jnp.min(jnp.where(probs == m1, eidx, E), axis=1, keepdims=True)
    sel1 = eidx == i1
    rest = jnp.where(sel1, -1.0, probs)
    m2 = jnp.max(rest, axis=1, keepdims=True)
    i2 = jnp.min(jnp.where(rest == m2, eidx, E), axis=1, keepdims=True)
    sel2 = eidx == i2
    sel = sel1 | sel2
    sel_f = sel.astype(jnp.float32)

    denom = m1 + m2 + 1e-8
    pnorm = jnp.where(sel1, m1 / denom, 0.0) + jnp.where(sel2, m2 / denom, 0.0)

    ih = jnp.maximum(_cdot(h, iw1_ref[...]) + ib1_ref[...], 0.0)
    il = _cdot(ih, iw2_ref[...]) + ib2_ref[...]
    imp = jax.nn.sigmoid(il)
    imp_ref[...] = imp
    factor = 1.0 + (imp > THRESH).astype(jnp.float32)

    r_iota = jax.lax.broadcasted_iota(jnp.int32, (SB, SB), 0)
    c_iota = jax.lax.broadcasted_iota(jnp.int32, (SB, SB), 1)
    tri = (r_iota >= c_iota).astype(jnp.float32)
    csum = jnp.dot(tri, sel_f, preferred_element_type=jnp.float32)
    pos_f = cnt_ref[...] + csum - sel_f
    cnt_ref[...] = cnt_ref[...] + csum[SB - 1:SB, :]
    pos = pos_f.astype(jnp.int32)

    posk = jnp.where(sel & (pos < CAP), pos, -1)

    cap_iota = jax.lax.broadcasted_iota(jnp.int32, (SB, E, CAP), 2)
    hit = (cap_iota == posk[:, :, None]).astype(jnp.float32)
    disp_ref[...] = hit
    comb_ref[...] = hit * (pnorm * factor)[:, :, None]

    psum_ref[...] = psum_ref[...] + jnp.sum(probs, axis=0, keepdims=True)
    usum_ref[...] = usum_ref[...] + csum[SB - 1:SB, :]

    @pl.when(i == nsteps - 1)
    def _fin():
        prob_mean = psum_ref[...] / S
        usage = usum_ref[...] / (S * K)
        aux_ref[...] = jnp.sum(prob_mean * usage,
                               keepdims=True).reshape(1, 1) * E


def kernel(hidden_states, r_w1, r_b1, r_w2, r_b2,
           imp_w1, imp_b1, imp_w2, imp_b2):
    B = hidden_states.shape[0]
    h2 = hidden_states.reshape(B * S, H)
    grid = (B * S) // SB

    out_shapes = (
        jax.ShapeDtypeStruct((B * S, E, CAP), jnp.float32),
        jax.ShapeDtypeStruct((B * S, E, CAP), jnp.float32),
        jax.ShapeDtypeStruct((B * S, E), jnp.float32),
        jax.ShapeDtypeStruct((B * S, 1), jnp.float32),
        jax.ShapeDtypeStruct((1, 1), jnp.float32),
    )
    full = lambda *shape: pl.BlockSpec(shape, lambda i: (0,) * len(shape))
    outs = pl.pallas_call(
        _router_body,
        grid=(grid,),
        in_specs=[
            pl.BlockSpec((SB, H), lambda i: (i, 0)),
            full(H, H),
            full(1, H),
            full(H, E),
            full(1, E),
            full(H, H // 2),
            full(1, H // 2),
            full(H // 2, 1),
            full(1, 1),
        ],
        out_specs=[
            pl.BlockSpec((SB, E, CAP), lambda i: (i, 0, 0)),
            pl.BlockSpec((SB, E, CAP), lambda i: (i, 0, 0)),
            pl.BlockSpec((SB, E), lambda i: (i, 0)),
            pl.BlockSpec((SB, 1), lambda i: (i, 0)),
            pl.BlockSpec((1, 1), lambda i: (0, 0)),
        ],
        out_shape=out_shapes,
        scratch_shapes=[
            pltpu.VMEM((1, E), jnp.float32),
            pltpu.VMEM((1, E), jnp.float32),
            pltpu.VMEM((1, E), jnp.float32),
        ],
    )(h2, r_w1, r_b1.reshape(1, H), r_w2, r_b2.reshape(1, E),
      imp_w1, imp_b1.reshape(1, H // 2), imp_w2, imp_b2.reshape(1, 1))

    disp, comb, probs, imp, aux = outs
    dispatch = disp.reshape(B, S, E, CAP)
    combine = comb.reshape(B, S, E, CAP)
    router_probs = probs.reshape(B, S, E)
    importance = imp.reshape(B, S)
    aux_loss = aux.reshape(())
    return (dispatch, combine, router_probs, aux_loss, importance)
